# baseline (speedup 1.0000x reference)
import jax
import jax.numpy as jnp
from jax.experimental import pallas as pl
from jax.experimental.pallas import tpu as pltpu

B, N, D, R = 4, 8192, 1024, 16
TN = 4096
SC = 512
NC = TN // SC
NT = N // TN
SUB = 64
LANE = 128


def _proj_body(x_ref, wcat_ref, s_ref, kb_scr, qg_scr):
    t = pl.program_id(1)
    xt = x_ref[0]
    dn = (((1,), (1,)), ((), ()))
    qk = jax.lax.dot_general(xt, wcat_ref[...], dn,
                             preferred_element_type=jnp.float32)
    q = qk[:, 0:R]
    kl = qk[:, R:2 * R]
    kb_scr[pl.ds(t * TN, TN), :] = kl.astype(jnp.bfloat16)
    part0 = jnp.sum(q[0:SC], axis=0, keepdims=True)
    acc = jnp.where(t == 0, part0, qg_scr[...] + part0)
    for c in range(1, NC):
        acc = acc + jnp.sum(q[c * SC:(c + 1) * SC], axis=0, keepdims=True)
    qg_scr[...] = acc

    @pl.when(t == NT - 1)
    def _():
        qg = qg_scr[...] * jnp.float32(1.0 / N)
        s_ref[0] = jax.lax.dot_general(qg, kb_scr[...], dn,
                                       preferred_element_type=jnp.float32)


def _partner(a, j, upper):
    if j < LANE:
        down, up = jnp.roll(a, j, axis=2), jnp.roll(a, -j, axis=2)
    else:
        down, up = jnp.roll(a, j // LANE, axis=1), jnp.roll(a, -(j // LANE), axis=1)
    return jnp.where(upper, down, up)


def _sort_body(s_ref, out_ref):
    s = s_ref[...]
    sb = jax.lax.bitcast_convert_type(s, jnp.int32)
    mono = sb ^ (jax.lax.shift_right_arithmetic(sb, 31) & jnp.int32(0x7FFFFFFF))
    key = ~mono
    pos = (jax.lax.broadcasted_iota(jnp.int32, (B, SUB, LANE), 1) * LANE
           + jax.lax.broadcasted_iota(jnp.int32, (B, SUB, LANE), 2))
    idx = pos

    k_sz = 2
    while k_sz <= N:
        j = k_sz // 2
        while j >= 1:
            upper = (pos & j) != 0
            pk = _partner(key, j, upper)
            pi = _partner(idx, j, upper)
            take_min = jnp.logical_not(
                jnp.logical_xor((pos & k_sz) == 0, jnp.logical_not(upper)))
            self_lt = (key < pk) | ((key == pk) & (idx < pi))
            keep_self = jnp.logical_not(jnp.logical_xor(self_lt, take_min))
            key = jnp.where(keep_self, key, pk)
            idx = jnp.where(keep_self, idx, pi)
            j //= 2
        k_sz *= 2

    out_ref[...] = idx[:, :8, :]


@jax.jit
def kernel(x, Wq, Wk):
    b, n, d = x.shape
    k = max(1, int(n * 0.1))

    wcat = jnp.concatenate([Wq, Wk], axis=0)

    scores = pl.pallas_call(
        _proj_body,
        grid=(B, NT),
        in_specs=[
            pl.BlockSpec((1, TN, D), lambda bb, t: (bb, t, 0)),
            pl.BlockSpec((2 * R, D), lambda bb, t: (0, 0)),
        ],
        out_specs=pl.BlockSpec((1, 1, N), lambda bb, t: (bb, 0, 0)),
        out_shape=jax.ShapeDtypeStruct((B, 1, N), jnp.float32),
        scratch_shapes=[
            pltpu.VMEM((N, R), jnp.bfloat16),
            pltpu.VMEM((1, R), jnp.float32),
        ],
    )(x, wcat)

    top = pl.pallas_call(
        _sort_body,
        out_shape=jax.ShapeDtypeStruct((B, 8, LANE), jnp.int32),
    )(scores.reshape(B, SUB, LANE))

    return top.reshape(B, 8 * LANE)[:, :k]


# speedup vs baseline: 1.0419x; 1.0419x over previous
import jax
import jax.numpy as jnp
from jax.experimental import pallas as pl
from jax.experimental.pallas import tpu as pltpu

B, N, D, R = 4, 8192, 1024, 16
TN = 2048
SC = 512
NC = TN // SC
NT = N // TN
SUB = 64
LANE = 128


def _proj_body(x_ref, wcat_ref, s_ref, kb_scr, qg_scr):
    t = pl.program_id(1)
    xt = x_ref[0]
    dn = (((1,), (1,)), ((), ()))
    qk = jax.lax.dot_general(xt, wcat_ref[...], dn,
                             preferred_element_type=jnp.float32)
    q = qk[:, 0:R]
    kl = qk[:, R:2 * R]
    kb_scr[pl.ds(t * TN, TN), :] = kl.astype(jnp.bfloat16)
    part0 = jnp.sum(q[0:SC], axis=0, keepdims=True)
    acc = jnp.where(t == 0, part0, qg_scr[...] + part0)
    for c in range(1, NC):
        acc = acc + jnp.sum(q[c * SC:(c + 1) * SC], axis=0, keepdims=True)
    qg_scr[...] = acc

    @pl.when(t == NT - 1)
    def _():
        qg = qg_scr[...] * jnp.float32(1.0 / N)
        s_ref[0] = jax.lax.dot_general(qg, kb_scr[...], dn,
                                       preferred_element_type=jnp.float32)


def _partner(a, j, upper):
    if j < LANE:
        down, up = jnp.roll(a, j, axis=2), jnp.roll(a, -j, axis=2)
    else:
        down, up = jnp.roll(a, j // LANE, axis=1), jnp.roll(a, -(j // LANE), axis=1)
    return jnp.where(upper, down, up)


def _stage(key, idx, pos, k_sz, j):
    upper = (pos & j) != 0
    pk = _partner(key, j, upper)
    pi = _partner(idx, j, upper)
    take_min = jnp.logical_not(
        jnp.logical_xor((pos & k_sz) == 0, jnp.logical_not(upper)))
    self_lt = (key < pk) | ((key == pk) & (idx < pi))
    keep_self = jnp.logical_not(jnp.logical_xor(self_lt, take_min))
    return jnp.where(keep_self, key, pk), jnp.where(keep_self, idx, pi)


def _iota_pos(rows):
    return (jax.lax.broadcasted_iota(jnp.int32, (B, rows, LANE), 1) * LANE
            + jax.lax.broadcasted_iota(jnp.int32, (B, rows, LANE), 2))


def _sort_body(s_ref, out_ref):
    s = s_ref[...]
    sb = jax.lax.bitcast_convert_type(s, jnp.int32)
    mono = sb ^ (jax.lax.shift_right_arithmetic(sb, 31) & jnp.int32(0x7FFFFFFF))
    key = ~mono
    pos = _iota_pos(SUB)
    idx = pos

    k_sz = 2
    while k_sz <= 1024:
        j = k_sz // 2
        while j >= 1:
            key, idx = _stage(key, idx, pos, k_sz, j)
            j //= 2
        k_sz *= 2

    rows = SUB
    while rows > 8:
        kparts, iparts = [], []
        for m in range(rows // 16):
            ka = key[:, 16 * m:16 * m + 8, :]
            kb_ = key[:, 16 * m + 8:16 * m + 16, :]
            ia = idx[:, 16 * m:16 * m + 8, :]
            ib = idx[:, 16 * m + 8:16 * m + 16, :]
            lt = (ka < kb_) | ((ka == kb_) & (ia < ib))
            kparts.append(jnp.where(lt, ka, kb_))
            iparts.append(jnp.where(lt, ia, ib))
        key = jnp.concatenate(kparts, axis=1)
        idx = jnp.concatenate(iparts, axis=1)
        rows //= 2
        pos = _iota_pos(rows)
        j = 512
        while j >= 1:
            key, idx = _stage(key, idx, pos, 1024, j)
            j //= 2

    out_ref[...] = idx


@jax.jit
def kernel(x, Wq, Wk):
    b, n, d = x.shape
    k = max(1, int(n * 0.1))

    wcat = jnp.concatenate([Wq, Wk], axis=0)

    scores = pl.pallas_call(
        _proj_body,
        grid=(B, NT),
        in_specs=[
            pl.BlockSpec((1, TN, D), lambda bb, t: (bb, t, 0)),
            pl.BlockSpec((2 * R, D), lambda bb, t: (0, 0)),
        ],
        out_specs=pl.BlockSpec((1, 1, N), lambda bb, t: (bb, 0, 0)),
        out_shape=jax.ShapeDtypeStruct((B, 1, N), jnp.float32),
        scratch_shapes=[
            pltpu.VMEM((N, R), jnp.bfloat16),
            pltpu.VMEM((1, R), jnp.float32),
        ],
    )(x, wcat)

    top = pl.pallas_call(
        _sort_body,
        out_shape=jax.ShapeDtypeStruct((B, 8, LANE), jnp.int32),
    )(scores.reshape(B, SUB, LANE))

    return top.reshape(B, 8 * LANE)[:, :k]
